# baseline (speedup 1.0000x reference)
import functools

import jax
import jax.numpy as jnp
from jax import lax
from jax.experimental import pallas as pl
from jax.experimental.pallas import tpu as pltpu
from jax.experimental.pallas import tpu_sc as plsc

BATCH = 16384
D0, D1, D2 = 16, 32, 64

_NC = 2
_NS = 16
_NW = _NC * _NS
_BPW = BATCH // _NW
_CHUNK = 128
_NCHUNK = _BPW // _CHUNK


def _relu_inplace(ref, d, rows_per_iter):
    nslice = d // 16

    def body(i, carry):
        base_row = i * rows_per_iter
        for r in range(rows_per_iter):
            for j in range(nslice):
                sl = pl.ds(j * 16, 16)
                row = base_row + r
                ref[row, sl] = jnp.maximum(ref[row, sl], 0.0)
        return carry

    lax.fori_loop(0, _BPW // rows_per_iter, body, 0)


def _sc_body(idx_hbm, w0, w1, w2, o0, o1, o2,
             idx_v, r0, r1, r2, s0, s1, s2, so):
    wid = lax.axis_index("s") * _NC + lax.axis_index("c")
    base = wid * _BPW

    pltpu.sync_copy(idx_hbm.at[pl.ds(wid * _NCHUNK, _NCHUNK)], idx_v)

    gathers = []
    for (w, r, s) in ((w0, r0, s0), (w1, r1, s1), (w2, r2, s2)):
        for j in range(_NCHUNK):
            gathers.append(pltpu.async_copy(
                w.at[idx_v.at[j]], r.at[pl.ds(j * _CHUNK, _CHUNK)], s))

    outs = []
    for (r, o, d, rows) in ((r0, o0, D0, 32), (r1, o1, D1, 16), (r2, o2, D2, 8)):
        for j in range(_NCHUNK):
            gathers.pop(0).wait()
        _relu_inplace(r, d, rows)
        outs.append(pltpu.async_copy(r, o.at[pl.ds(base, _BPW)], so))

    for c in outs:
        c.wait()


_gather_relu = pl.kernel(
    _sc_body,
    out_type=(
        jax.ShapeDtypeStruct((BATCH, D0), jnp.float32),
        jax.ShapeDtypeStruct((BATCH, D1), jnp.float32),
        jax.ShapeDtypeStruct((BATCH, D2), jnp.float32),
    ),
    mesh=plsc.VectorSubcoreMesh(core_axis_name="c", subcore_axis_name="s"),
    compiler_params=pltpu.CompilerParams(use_tc_tiling_on_sc=False),
    scratch_types=[
        pltpu.VMEM((_NCHUNK, _CHUNK), jnp.int32),
        pltpu.VMEM((_BPW, D0), jnp.float32),
        pltpu.VMEM((_BPW, D1), jnp.float32),
        pltpu.VMEM((_BPW, D2), jnp.float32),
        pltpu.SemaphoreType.DMA,
        pltpu.SemaphoreType.DMA,
        pltpu.SemaphoreType.DMA,
        pltpu.SemaphoreType.DMA,
    ],
)


def kernel(indices, W0, W1, W2):
    idx = indices.astype(jnp.int32).reshape(_NW * _NCHUNK, _CHUNK)
    return _gather_relu(idx, W0, W1, W2)
